# Initial kernel scaffold; baseline (speedup 1.0000x reference)
import jax
import jax.numpy as jnp
from jax.experimental import pallas as pl


def kernel(x, edge_index, W1, b1, g1, be1, W2, b2, g2, be2):
    raise NotImplementedError("write your pallas kernel here")


# speedup vs baseline: 8.1015x; 8.1015x over previous
import functools

import jax
import jax.numpy as jnp
from jax import lax
from jax.experimental import pallas as pl
from jax.experimental.pallas import tpu as pltpu
from jax.experimental.pallas import tpu_sc as plsc

N = 10000
E = 320000
D = 128
NW = 32
LB = 128
BPW = 80
NBP = NW * BPW
EPAD = NBP * LB - E
K = 2
ITERS = BPW // K
NP = 10240
RPT = NP // 16


def _segment_sum_sc(x, src_b, dst_b):
    mesh = plsc.VectorSubcoreMesh(core_axis_name="c", subcore_axis_name="s")

    @functools.partial(
        pl.kernel,
        mesh=mesh,
        out_type=jax.ShapeDtypeStruct((2 * NP, D), jnp.float32),
        scratch_types=[
            pltpu.VMEM((K, LB), jnp.int32),
            pltpu.VMEM((K, LB), jnp.int32),
            pltpu.VMEM((K, LB, D), jnp.float32),
            pltpu.VMEM((16, D), jnp.float32),
            pltpu.VMEM_SHARED((NP, D), jnp.float32),
            pltpu.SemaphoreType.DMA,
        ],
    )
    def seg_kernel(x_hbm, srcb_hbm, dstb_hbm, out_hbm, sidx, didx, rows, zbuf, agg, sem):
        c = lax.axis_index("c")
        s = lax.axis_index("s")
        wid = c * 16 + s

        for r in range(16):
            for q in range(D // 16):
                zbuf[r, pl.ds(q * 16, 16)] = jnp.zeros((16,), jnp.float32)

        def zero_body(i, carry):
            pltpu.sync_copy(zbuf, agg.at[pl.ds(s * RPT + i * 16, 16)])
            return carry

        lax.fori_loop(0, RPT // 16, zero_body, 0)
        plsc.subcore_barrier()

        def body(i, carry):
            b0 = wid * BPW + i * K
            pltpu.sync_copy(srcb_hbm.at[pl.ds(b0, K)], sidx)
            pltpu.sync_copy(dstb_hbm.at[pl.ds(b0, K)], didx)
            cps = [
                pltpu.async_copy(x_hbm.at[sidx.at[j]], rows.at[j], sem)
                for j in range(K)
            ]
            for cp in cps:
                cp.wait()
            for j in range(K):
                pltpu.sync_copy(rows.at[j], agg.at[didx.at[j]], add=True)
            return carry

        lax.fori_loop(0, ITERS, body, 0)
        plsc.subcore_barrier()

        pltpu.sync_copy(
            agg.at[pl.ds(s * RPT, RPT)],
            out_hbm.at[pl.ds(c * NP + s * RPT, RPT)],
        )

    return seg_kernel(x, src_b, dst_b)


def _mlp_body(x_ref, p_ref, w1_ref, b1_ref, g1_ref, be1_ref,
              w2_ref, b2_ref, g2_ref, be2_ref, o_ref):
    h = x_ref[...] + p_ref[0:N, :] + p_ref[NP:NP + N, :]
    h1 = jnp.dot(h, w1_ref[...], preferred_element_type=jnp.float32) + b1_ref[...]
    h1 = jnp.maximum(h1, 0.0)
    m1 = jnp.mean(h1, axis=0, keepdims=True)
    v1 = jnp.mean((h1 - m1) * (h1 - m1), axis=0, keepdims=True)
    h1 = (h1 - m1) * lax.rsqrt(v1 + 1e-5) * g1_ref[...] + be1_ref[...]
    h2 = jnp.dot(h1, w2_ref[...], preferred_element_type=jnp.float32) + b2_ref[...]
    m2 = jnp.mean(h2, axis=0, keepdims=True)
    v2 = jnp.mean((h2 - m2) * (h2 - m2), axis=0, keepdims=True)
    h2 = (h2 - m2) * lax.rsqrt(v2 + 1e-5) * g2_ref[...] + be2_ref[...]
    o_ref[...] = jnp.where(h2 >= 0, h2, 0.01 * h2)


def kernel(x, edge_index, W1, b1, g1, be1, W2, b2, g2, be2):
    src = edge_index[0]
    dst = edge_index[1]
    pad_iota = lax.iota(jnp.int32, EPAD)
    src_p = jnp.concatenate([src, (pad_iota * 997) % N])
    dst_p = jnp.concatenate([dst, N + pad_iota % (NP - N)])
    src_b = src_p.reshape(NBP, LB)
    dst_b = dst_p.reshape(NBP, LB)

    partials = _segment_sum_sc(x, src_b, dst_b)

    out = pl.pallas_call(
        _mlp_body,
        out_shape=jax.ShapeDtypeStruct((N, 64), jnp.float32),
    )(
        x, partials, W1,
        b1.reshape(1, -1), g1.reshape(1, -1), be1.reshape(1, -1),
        W2,
        b2.reshape(1, -1), g2.reshape(1, -1), be2.reshape(1, -1),
    )
    return out


# speedup vs baseline: 10.5217x; 1.2987x over previous
import functools

import jax
import jax.numpy as jnp
from jax import lax
from jax.experimental import pallas as pl
from jax.experimental.pallas import tpu as pltpu
from jax.experimental.pallas import tpu_sc as plsc

N = 10000
E = 320000
D = 128
NW = 32
LB = 128
BPW = 80
EPAD = NW * BPW * LB - E
SLOP = 2
SL = BPW + SLOP
NBP = NW * SL
NP = 10240
RPT = NP // 16


def _segment_sum_sc(x, idx_all):
    mesh = plsc.VectorSubcoreMesh(core_axis_name="c", subcore_axis_name="s")

    @functools.partial(
        pl.kernel,
        mesh=mesh,
        out_type=jax.ShapeDtypeStruct((2 * NP, D), jnp.float32),
        scratch_types=[
            pltpu.VMEM((2, 1, 2, LB), jnp.int32),
            pltpu.VMEM((2, LB, D), jnp.float32),
            pltpu.VMEM((16, D), jnp.float32),
            pltpu.VMEM_SHARED((NP, D), jnp.float32),
            pltpu.SemaphoreType.DMA,
            pltpu.SemaphoreType.DMA,
        ],
    )
    def seg_kernel(x_hbm, idx_hbm, out_hbm, ibuf, rows, zbuf, agg, gsem, isem):
        c = lax.axis_index("c")
        s = lax.axis_index("s")
        wid = c * 16 + s
        base = wid * SL

        def fire_idx(b, slot):
            return pltpu.async_copy(
                idx_hbm.at[pl.ds(base + b, 1)], ibuf.at[slot], isem)

        def wait_idx(slot):
            pltpu.make_async_copy(
                idx_hbm.at[pl.ds(base, 1)], ibuf.at[slot], isem).wait()

        def fire_gather(slot):
            return pltpu.async_copy(
                x_hbm.at[ibuf.at[slot, 0, 0]], rows.at[slot], gsem)

        def wait_gather(slot):
            pltpu.make_async_copy(
                x_hbm.at[ibuf.at[slot, 0, 0]], rows.at[slot], gsem).wait()

        fire_idx(0, 0)
        fire_idx(1, 1)

        for r in range(16):
            for q in range(D // 16):
                zbuf[r, pl.ds(q * 16, 16)] = jnp.zeros((16,), jnp.float32)

        def zero_body(i, carry):
            pltpu.sync_copy(zbuf, agg.at[pl.ds(s * RPT + i * 16, 16)])
            return carry

        lax.fori_loop(0, RPT // 16, zero_body, 0)

        wait_idx(0)
        fire_gather(0)
        plsc.subcore_barrier()

        def body(i, carry):
            for j in (0, 1):
                b = i * 2 + j
                wait_idx(1 - j)
                wait_gather(j)
                fire_gather(1 - j)
                pltpu.sync_copy(rows.at[j], agg.at[ibuf.at[j, 0, 1]], add=True)
                fire_idx(b + 2, j)
            return carry

        lax.fori_loop(0, BPW // 2, body, 0)

        wait_idx(1)
        wait_gather(0)
        plsc.subcore_barrier()

        pltpu.sync_copy(
            agg.at[pl.ds(s * RPT, RPT)],
            out_hbm.at[pl.ds(c * NP + s * RPT, RPT)],
        )

    return seg_kernel(x, idx_all)


def _mlp_body(x_ref, p_ref, w1_ref, b1_ref, g1_ref, be1_ref,
              w2_ref, b2_ref, g2_ref, be2_ref, o_ref):
    h = x_ref[...] + p_ref[0:N, :] + p_ref[NP:NP + N, :]
    h1 = jnp.dot(h, w1_ref[...], preferred_element_type=jnp.float32) + b1_ref[...]
    h1 = jnp.maximum(h1, 0.0)
    m1 = jnp.mean(h1, axis=0, keepdims=True)
    v1 = jnp.mean((h1 - m1) * (h1 - m1), axis=0, keepdims=True)
    h1 = (h1 - m1) * lax.rsqrt(v1 + 1e-5) * g1_ref[...] + be1_ref[...]
    h2 = jnp.dot(h1, w2_ref[...], preferred_element_type=jnp.float32) + b2_ref[...]
    m2 = jnp.mean(h2, axis=0, keepdims=True)
    v2 = jnp.mean((h2 - m2) * (h2 - m2), axis=0, keepdims=True)
    h2 = (h2 - m2) * lax.rsqrt(v2 + 1e-5) * g2_ref[...] + be2_ref[...]
    o_ref[...] = jnp.where(h2 >= 0, h2, 0.01 * h2)


def kernel(x, edge_index, W1, b1, g1, be1, W2, b2, g2, be2):
    src = edge_index[0]
    dst = edge_index[1]
    pad_iota = lax.iota(jnp.int32, EPAD)
    src_p = jnp.concatenate([src, (pad_iota * 997) % N])
    dst_p = jnp.concatenate([dst, N + pad_iota % (NP - N)])
    idx = jnp.stack([src_p.reshape(NW * BPW, LB), dst_p.reshape(NW * BPW, LB)],
                    axis=1).reshape(NW, BPW, 2, LB)
    slop_iota = lax.iota(jnp.int32, NW * SLOP * 2 * LB)
    slop = ((slop_iota * 79) % N).reshape(NW, SLOP, 2, LB)
    idx_all = jnp.concatenate([idx, slop], axis=1).reshape(NBP, 2, LB)

    partials = _segment_sum_sc(x, idx_all)

    out = pl.pallas_call(
        _mlp_body,
        out_shape=jax.ShapeDtypeStruct((N, 64), jnp.float32),
    )(
        x, partials, W1,
        b1.reshape(1, -1), g1.reshape(1, -1), be1.reshape(1, -1),
        W2,
        b2.reshape(1, -1), g2.reshape(1, -1), be2.reshape(1, -1),
    )
    return out


# speedup vs baseline: 11.1279x; 1.0576x over previous
import functools

import jax
import jax.numpy as jnp
from jax import lax
from jax.experimental import pallas as pl
from jax.experimental.pallas import tpu as pltpu
from jax.experimental.pallas import tpu_sc as plsc

N = 10000
E = 320000
D = 128
NW = 32
LB = 128
BPW = 81
EPAD = NW * BPW * LB - E
SLOP = 3
SL = BPW + SLOP
NBP = NW * SL
NP = 10112
RPT = NP // 16


def _segment_sum_sc(x, idx_all):
    mesh = plsc.VectorSubcoreMesh(core_axis_name="c", subcore_axis_name="s")

    @functools.partial(
        pl.kernel,
        mesh=mesh,
        out_type=jax.ShapeDtypeStruct((2 * NP, D), jnp.float32),
        scratch_types=[
            pltpu.VMEM((3, 1, 2, LB), jnp.int32),
            pltpu.VMEM((3, LB, D), jnp.float32),
            pltpu.VMEM_SHARED((NP, D), jnp.float32),
            pltpu.SemaphoreType.DMA,
            pltpu.SemaphoreType.DMA,
        ],
    )
    def seg_kernel(x_hbm, idx_hbm, out_hbm, ibuf, rows, agg, gsem, isem):
        c = lax.axis_index("c")
        s = lax.axis_index("s")
        wid = c * 16 + s
        base = wid * SL

        def fire_idx(b, slot):
            return pltpu.async_copy(
                idx_hbm.at[pl.ds(base + b, 1)], ibuf.at[slot], isem)

        def wait_idx(slot):
            pltpu.make_async_copy(
                idx_hbm.at[pl.ds(base, 1)], ibuf.at[slot], isem).wait()

        def fire_gather(slot):
            return pltpu.async_copy(
                x_hbm.at[ibuf.at[slot, 0, 0]], rows.at[slot], gsem)

        def wait_gather(slot):
            pltpu.make_async_copy(
                x_hbm.at[ibuf.at[slot, 0, 0]], rows.at[slot], gsem).wait()

        fire_idx(0, 0)
        fire_idx(1, 1)
        fire_idx(2, 2)

        def zrow(r, carry):
            for k in (0, 1, 2):
                for q in range(D // 16):
                    rows[k, r, pl.ds(q * 16, 16)] = jnp.zeros((16,), jnp.float32)
            return carry

        lax.fori_loop(0, LB, zrow, 0)
        zcps = [
            pltpu.async_copy(rows.at[k % 3],
                             agg.at[pl.ds(s * RPT + k * LB, LB)], gsem)
            for k in range(4)
        ]
        zcps.append(pltpu.async_copy(
            rows.at[1].at[pl.ds(0, RPT - 4 * LB)],
            agg.at[pl.ds(s * RPT + 4 * LB, RPT - 4 * LB)], gsem))
        for cp in zcps:
            cp.wait()

        wait_idx(0)
        fire_gather(0)
        wait_idx(1)
        fire_gather(1)
        plsc.subcore_barrier()

        def body(i, carry):
            for j in (0, 1, 2):
                b = i * 3 + j
                wait_idx((b + 2) % 3)
                fire_gather((b + 2) % 3)
                wait_gather(b % 3)
                pltpu.sync_copy(rows.at[b % 3],
                                agg.at[ibuf.at[b % 3, 0, 1]], add=True)
                fire_idx(b + 3, b % 3)
            return carry

        lax.fori_loop(0, BPW // 3, body, 0)

        wait_gather(0)
        wait_gather(1)
        wait_idx(2)
        plsc.subcore_barrier()

        pltpu.sync_copy(
            agg.at[pl.ds(s * RPT, RPT)],
            out_hbm.at[pl.ds(c * NP + s * RPT, RPT)],
        )

    return seg_kernel(x, idx_all)


def _mlp_body(x_ref, p_ref, w1_ref, b1_ref, g1_ref, be1_ref,
              w2_ref, b2_ref, g2_ref, be2_ref, o_ref):
    h = x_ref[...] + p_ref[0:N, :] + p_ref[NP:NP + N, :]
    h1 = jnp.dot(h, w1_ref[...], preferred_element_type=jnp.float32) + b1_ref[...]
    h1 = jnp.maximum(h1, 0.0)
    m1 = jnp.mean(h1, axis=0, keepdims=True)
    v1 = jnp.mean((h1 - m1) * (h1 - m1), axis=0, keepdims=True)
    h1 = (h1 - m1) * lax.rsqrt(v1 + 1e-5) * g1_ref[...] + be1_ref[...]
    h2 = jnp.dot(h1, w2_ref[...], preferred_element_type=jnp.float32) + b2_ref[...]
    m2 = jnp.mean(h2, axis=0, keepdims=True)
    v2 = jnp.mean((h2 - m2) * (h2 - m2), axis=0, keepdims=True)
    h2 = (h2 - m2) * lax.rsqrt(v2 + 1e-5) * g2_ref[...] + be2_ref[...]
    o_ref[...] = jnp.where(h2 >= 0, h2, 0.01 * h2)


def kernel(x, edge_index, W1, b1, g1, be1, W2, b2, g2, be2):
    src = edge_index[0]
    dst = edge_index[1]
    pad_iota = lax.iota(jnp.int32, EPAD)
    src_p = jnp.concatenate([src, (pad_iota * 997) % N])
    dst_p = jnp.concatenate([dst, N + pad_iota % (NP - N)])
    idx = jnp.stack([src_p.reshape(NW * BPW, LB), dst_p.reshape(NW * BPW, LB)],
                    axis=1).reshape(NW, BPW, 2, LB)
    slop_iota = lax.iota(jnp.int32, NW * SLOP * 2 * LB)
    slop = ((slop_iota * 79) % N).reshape(NW, SLOP, 2, LB)
    idx_all = jnp.concatenate([idx, slop], axis=1).reshape(NBP, 2, LB)

    partials = _segment_sum_sc(x, idx_all)

    out = pl.pallas_call(
        _mlp_body,
        out_shape=jax.ShapeDtypeStruct((N, 64), jnp.float32),
    )(
        x, partials, W1,
        b1.reshape(1, -1), g1.reshape(1, -1), be1.reshape(1, -1),
        W2,
        b2.reshape(1, -1), g2.reshape(1, -1), be2.reshape(1, -1),
    )
    return out


# speedup vs baseline: 11.4212x; 1.0264x over previous
import functools

import jax
import jax.numpy as jnp
import numpy as np
from jax import lax
from jax.experimental import pallas as pl
from jax.experimental.pallas import tpu as pltpu
from jax.experimental.pallas import tpu_sc as plsc

N = 10000
E = 320000
D = 128
NW = 32
LB = 128
BPW = 81
SLOP = 3
NE_P = (NW * BPW + SLOP) * LB
NP = 10112
RPT = NP // 16

_PAD_SRC = np.asarray((np.arange(NE_P - E) * 997) % N, dtype=np.int32)
_PAD_DST = np.asarray(N + np.arange(NE_P - E) % (NP - N), dtype=np.int32)


def _segment_sum_sc(x, src_p, dst_p):
    mesh = plsc.VectorSubcoreMesh(core_axis_name="c", subcore_axis_name="s")

    @functools.partial(
        pl.kernel,
        mesh=mesh,
        out_type=jax.ShapeDtypeStruct((2 * NP, D), jnp.float32),
        scratch_types=[
            pltpu.VMEM((3, 1, 2, LB), jnp.int32),
            pltpu.VMEM((3, LB, D), jnp.float32),
            pltpu.VMEM_SHARED((NP, D), jnp.float32),
            pltpu.SemaphoreType.DMA,
            pltpu.SemaphoreType.DMA,
        ],
    )
    def seg_kernel(x_hbm, src_hbm, dst_hbm, out_hbm, ibuf, rows, agg, gsem, isem):
        c = lax.axis_index("c")
        s = lax.axis_index("s")
        wid = c * 16 + s
        base = wid * BPW * LB

        def fire_idx(b, slot):
            off = base + b * LB
            pltpu.async_copy(src_hbm.at[pl.ds(off, LB)], ibuf.at[slot, 0, 0], isem)
            pltpu.async_copy(dst_hbm.at[pl.ds(off, LB)], ibuf.at[slot, 0, 1], isem)

        def wait_idx(slot):
            pltpu.make_async_copy(
                src_hbm.at[pl.ds(base, LB)], ibuf.at[slot, 0, 0], isem).wait()
            pltpu.make_async_copy(
                dst_hbm.at[pl.ds(base, LB)], ibuf.at[slot, 0, 1], isem).wait()

        def fire_gather(slot):
            return pltpu.async_copy(
                x_hbm.at[ibuf.at[slot, 0, 0]], rows.at[slot], gsem)

        def wait_gather(slot):
            pltpu.make_async_copy(
                x_hbm.at[ibuf.at[slot, 0, 0]], rows.at[slot], gsem).wait()

        fire_idx(0, 0)
        fire_idx(1, 1)
        fire_idx(2, 2)

        def zrow(r, carry):
            for k in (0, 1, 2):
                for q in range(D // 16):
                    rows[k, r, pl.ds(q * 16, 16)] = jnp.zeros((16,), jnp.float32)
            return carry

        lax.fori_loop(0, LB, zrow, 0)
        zcps = [
            pltpu.async_copy(rows.at[k % 3],
                             agg.at[pl.ds(s * RPT + k * LB, LB)], gsem)
            for k in range(4)
        ]
        zcps.append(pltpu.async_copy(
            rows.at[1].at[pl.ds(0, RPT - 4 * LB)],
            agg.at[pl.ds(s * RPT + 4 * LB, RPT - 4 * LB)], gsem))
        for cp in zcps:
            cp.wait()

        wait_idx(0)
        fire_gather(0)
        wait_idx(1)
        fire_gather(1)
        plsc.subcore_barrier()

        def body(i, carry):
            for j in (0, 1, 2):
                b = i * 3 + j
                wait_idx((b + 2) % 3)
                fire_gather((b + 2) % 3)
                wait_gather(b % 3)
                pltpu.sync_copy(rows.at[b % 3],
                                agg.at[ibuf.at[b % 3, 0, 1]], add=True)
                fire_idx(b + 3, b % 3)
            return carry

        lax.fori_loop(0, BPW // 3, body, 0)

        wait_gather(0)
        wait_gather(1)
        wait_idx(2)
        plsc.subcore_barrier()

        pltpu.sync_copy(
            agg.at[pl.ds(s * RPT, RPT)],
            out_hbm.at[pl.ds(c * NP + s * RPT, RPT)],
        )

    return seg_kernel(x, src_p, dst_p)


def _mlp_body(x_ref, p_ref, w1_ref, b1_ref, g1_ref, be1_ref,
              w2_ref, b2_ref, g2_ref, be2_ref, o_ref):
    h = x_ref[...] + p_ref[0:N, :] + p_ref[NP:NP + N, :]
    h1 = jnp.dot(h, w1_ref[...], preferred_element_type=jnp.float32) + b1_ref[...]
    h1 = jnp.maximum(h1, 0.0)
    m1 = jnp.mean(h1, axis=0, keepdims=True)
    v1 = jnp.mean((h1 - m1) * (h1 - m1), axis=0, keepdims=True)
    h1 = (h1 - m1) * lax.rsqrt(v1 + 1e-5) * g1_ref[...] + be1_ref[...]
    h2 = jnp.dot(h1, w2_ref[...], preferred_element_type=jnp.float32) + b2_ref[...]
    m2 = jnp.mean(h2, axis=0, keepdims=True)
    v2 = jnp.mean((h2 - m2) * (h2 - m2), axis=0, keepdims=True)
    h2 = (h2 - m2) * lax.rsqrt(v2 + 1e-5) * g2_ref[...] + be2_ref[...]
    o_ref[...] = jnp.where(h2 >= 0, h2, 0.01 * h2)


def kernel(x, edge_index, W1, b1, g1, be1, W2, b2, g2, be2):
    src_p = jnp.concatenate([edge_index[0], jnp.asarray(_PAD_SRC)])
    dst_p = jnp.concatenate([edge_index[1], jnp.asarray(_PAD_DST)])

    partials = _segment_sum_sc(x, src_p, dst_p)

    out = pl.pallas_call(
        _mlp_body,
        out_shape=jax.ShapeDtypeStruct((N, 64), jnp.float32),
    )(
        x, partials, W1,
        b1.reshape(1, -1), g1.reshape(1, -1), be1.reshape(1, -1),
        W2,
        b2.reshape(1, -1), g2.reshape(1, -1), be2.reshape(1, -1),
    )
    return out


# speedup vs baseline: 12.2124x; 1.0693x over previous
import functools

import jax
import jax.numpy as jnp
import numpy as np
from jax import lax
from jax.experimental import pallas as pl
from jax.experimental.pallas import tpu as pltpu
from jax.experimental.pallas import tpu_sc as plsc

N = 10000
E = 320000
D = 128
NW = 32
LB = 128
BPW = 81
SLOP = 3
NE_P = (NW * BPW + SLOP) * LB
NP = 10112
RPT = NP // 16

_PAD_SRC = np.asarray((np.arange(NE_P - E) * 997) % N, dtype=np.int32)
_PAD_DST = np.asarray(N + np.arange(NE_P - E) % (NP - N), dtype=np.int32)


def _segment_sum_sc(x, edge_index, pad_src, pad_dst):
    mesh = plsc.VectorSubcoreMesh(core_axis_name="c", subcore_axis_name="s")

    @functools.partial(
        pl.kernel,
        mesh=mesh,
        out_type=jax.ShapeDtypeStruct((2 * NP, D), jnp.float32),
        scratch_types=[
            pltpu.VMEM((3, 1, 2, LB), jnp.int32),
            pltpu.VMEM((3, LB, D), jnp.float32),
            pltpu.VMEM_SHARED((NP, D), jnp.float32),
            pltpu.SemaphoreType.DMA,
            pltpu.SemaphoreType.DMA,
        ],
    )
    def seg_kernel(x_hbm, edge_hbm, psrc_hbm, pdst_hbm, out_hbm,
                   ibuf, rows, agg, gsem, isem):
        c = lax.axis_index("c")
        s = lax.axis_index("s")
        wid = c * 16 + s
        base = wid * BPW * LB

        def fire_idx(b, slot):
            off = base + b * LB

            @pl.when(off < E)
            def _():
                pltpu.async_copy(edge_hbm.at[0, pl.ds(off, LB)],
                                 ibuf.at[slot, 0, 0], isem)
                pltpu.async_copy(edge_hbm.at[1, pl.ds(off, LB)],
                                 ibuf.at[slot, 0, 1], isem)

            @pl.when(off >= E)
            def _():
                pltpu.async_copy(psrc_hbm.at[pl.ds(off - E, LB)],
                                 ibuf.at[slot, 0, 0], isem)
                pltpu.async_copy(pdst_hbm.at[pl.ds(off - E, LB)],
                                 ibuf.at[slot, 0, 1], isem)

        def wait_idx(slot):
            pltpu.make_async_copy(
                psrc_hbm.at[pl.ds(0, LB)], ibuf.at[slot, 0, 0], isem).wait()
            pltpu.make_async_copy(
                pdst_hbm.at[pl.ds(0, LB)], ibuf.at[slot, 0, 1], isem).wait()

        def fire_gather(slot):
            return pltpu.async_copy(
                x_hbm.at[ibuf.at[slot, 0, 0]], rows.at[slot], gsem)

        def wait_gather(slot):
            pltpu.make_async_copy(
                x_hbm.at[ibuf.at[slot, 0, 0]], rows.at[slot], gsem).wait()

        fire_idx(0, 0)
        fire_idx(1, 1)
        fire_idx(2, 2)

        def zrow(r, carry):
            for k in (0, 1, 2):
                for q in range(D // 16):
                    rows[k, r, pl.ds(q * 16, 16)] = jnp.zeros((16,), jnp.float32)
            return carry

        lax.fori_loop(0, LB, zrow, 0)
        zcps = [
            pltpu.async_copy(rows.at[k % 3],
                             agg.at[pl.ds(s * RPT + k * LB, LB)], gsem)
            for k in range(4)
        ]
        zcps.append(pltpu.async_copy(
            rows.at[1].at[pl.ds(0, RPT - 4 * LB)],
            agg.at[pl.ds(s * RPT + 4 * LB, RPT - 4 * LB)], gsem))
        for cp in zcps:
            cp.wait()

        wait_idx(0)
        fire_gather(0)
        wait_idx(1)
        fire_gather(1)
        plsc.subcore_barrier()

        def body(i, carry):
            for j in (0, 1, 2):
                b = i * 3 + j
                wait_idx((b + 2) % 3)
                fire_gather((b + 2) % 3)
                wait_gather(b % 3)
                pltpu.sync_copy(rows.at[b % 3],
                                agg.at[ibuf.at[b % 3, 0, 1]], add=True)
                fire_idx(b + 3, b % 3)
            return carry

        lax.fori_loop(0, BPW // 3, body, 0)

        wait_gather(0)
        wait_gather(1)
        wait_idx(2)
        plsc.subcore_barrier()

        pltpu.sync_copy(
            agg.at[pl.ds(s * RPT, RPT)],
            out_hbm.at[pl.ds(c * NP + s * RPT, RPT)],
        )

    return seg_kernel(x, edge_index, pad_src, pad_dst)


def _mlp_body(x_ref, p_ref, w1_ref, b1_ref, g1_ref, be1_ref,
              w2_ref, b2_ref, g2_ref, be2_ref, o_ref):
    h = x_ref[...] + p_ref[0:N, :] + p_ref[NP:NP + N, :]
    h1 = jnp.dot(h, w1_ref[...], preferred_element_type=jnp.float32) + b1_ref[...]
    h1 = jnp.maximum(h1, 0.0)
    m1 = jnp.mean(h1, axis=0, keepdims=True)
    v1 = jnp.mean((h1 - m1) * (h1 - m1), axis=0, keepdims=True)
    h1 = (h1 - m1) * lax.rsqrt(v1 + 1e-5) * g1_ref[...] + be1_ref[...]
    h2 = jnp.dot(h1, w2_ref[...], preferred_element_type=jnp.float32) + b2_ref[...]
    m2 = jnp.mean(h2, axis=0, keepdims=True)
    v2 = jnp.mean((h2 - m2) * (h2 - m2), axis=0, keepdims=True)
    h2 = (h2 - m2) * lax.rsqrt(v2 + 1e-5) * g2_ref[...] + be2_ref[...]
    o_ref[...] = jnp.where(h2 >= 0, h2, 0.01 * h2)


def kernel(x, edge_index, W1, b1, g1, be1, W2, b2, g2, be2):
    partials = _segment_sum_sc(
        x, edge_index, jnp.asarray(_PAD_SRC), jnp.asarray(_PAD_DST))

    out = pl.pallas_call(
        _mlp_body,
        out_shape=jax.ShapeDtypeStruct((N, 64), jnp.float32),
    )(
        x, partials, W1,
        b1.reshape(1, -1), g1.reshape(1, -1), be1.reshape(1, -1),
        W2,
        b2.reshape(1, -1), g2.reshape(1, -1), be2.reshape(1, -1),
    )
    return out
